# Initial kernel scaffold; baseline (speedup 1.0000x reference)
import jax
import jax.numpy as jnp
from jax.experimental import pallas as pl


def kernel(score_norm_data, W1, W3, W2, Wg, A1, B1, A3, B3, A2, B2):
    raise NotImplementedError("write your pallas kernel here")


# speedup vs baseline: 1.7559x; 1.7559x over previous
import functools

import jax
import jax.numpy as jnp
from jax.experimental import pallas as pl
from jax.experimental.pallas import tpu as pltpu

N = 2048
D = 768
DFF = 2048
E = 8
R = 16

BT = 256
NT = N // BT


def _dot_t(a, b):
    return jax.lax.dot_general(a, b, (((1,), (1,)), ((), ())),
                               preferred_element_type=jnp.float32)


def _routing_weight(x_blk, Wg, e_idx):
    logits = _dot_t(x_blk, Wg)
    idx = jax.lax.broadcasted_iota(jnp.int32, logits.shape, 1)
    m1 = jnp.max(logits, axis=1, keepdims=True)
    i1 = jnp.min(jnp.where(logits == m1, idx, E), axis=1, keepdims=True)
    l2 = jnp.where(idx == i1, -jnp.inf, logits)
    m2 = jnp.max(l2, axis=1, keepdims=True)
    i2 = jnp.min(jnp.where(l2 == m2, idx, E), axis=1, keepdims=True)
    w1 = jax.nn.sigmoid(m1 - m2)
    we = jnp.where(i1 == e_idx, w1, jnp.where(i2 == e_idx, 1.0 - w1, 0.0))
    return we


def _ffn_kernel(x_ref, w1_ref, w3_ref, w2_ref, wg_ref,
                a1_ref, b1_ref, a3_ref, b3_ref, a2_ref, b2_ref,
                out_ref, cw1_ref, cw3_ref):
    e = pl.program_id(1)
    x = x_ref[...]

    @pl.when(e == 0)
    def _():
        cw1_ref[...] = _dot_t(x, w1_ref[...])
        cw3_ref[...] = _dot_t(x, w3_ref[...])
        out_ref[...] = jnp.zeros_like(out_ref)

    a1 = a1_ref[0]
    b1 = b1_ref[0]
    a3 = a3_ref[0]
    b3 = b3_ref[0]
    a2 = a2_ref[0]
    b2 = b2_ref[0]

    u1 = _dot_t(x, a1)
    w1 = cw1_ref[...] + _dot_t(u1, b1)
    u3 = _dot_t(x, a3)
    w3 = cw3_ref[...] + _dot_t(u3, b3)
    h = w1 * jax.nn.sigmoid(w1) * w3
    u2 = _dot_t(h, a2)
    hidden = _dot_t(h, w2_ref[...]) + _dot_t(u2, b2)

    we = _routing_weight(x, wg_ref[...], e)
    out_ref[...] += hidden * we


@jax.jit
def kernel(score_norm_data, W1, W3, W2, Wg, A1, B1, A3, B3, A2, B2):
    x = score_norm_data
    grid = (NT, E)
    out = pl.pallas_call(
        _ffn_kernel,
        grid=grid,
        in_specs=[
            pl.BlockSpec((BT, D), lambda t, e: (t, 0)),
            pl.BlockSpec((DFF, D), lambda t, e: (0, 0)),
            pl.BlockSpec((DFF, D), lambda t, e: (0, 0)),
            pl.BlockSpec((D, DFF), lambda t, e: (0, 0)),
            pl.BlockSpec((E, D), lambda t, e: (0, 0)),
            pl.BlockSpec((1, R, D), lambda t, e: (e, 0, 0)),
            pl.BlockSpec((1, DFF, R), lambda t, e: (e, 0, 0)),
            pl.BlockSpec((1, R, D), lambda t, e: (e, 0, 0)),
            pl.BlockSpec((1, DFF, R), lambda t, e: (e, 0, 0)),
            pl.BlockSpec((1, R, DFF), lambda t, e: (e, 0, 0)),
            pl.BlockSpec((1, D, R), lambda t, e: (e, 0, 0)),
        ],
        out_specs=pl.BlockSpec((BT, D), lambda t, e: (t, 0)),
        out_shape=jax.ShapeDtypeStruct((N, D), jnp.float32),
        scratch_shapes=[
            pltpu.VMEM((BT, DFF), jnp.float32),
            pltpu.VMEM((BT, DFF), jnp.float32),
        ],
        compiler_params=pltpu.CompilerParams(
            dimension_semantics=("parallel", "arbitrary"),
        ),
    )(x, W1, W3, W2, Wg, A1, B1, A3, B3, A2, B2)
    return out
